# baseline (speedup 1.0000x reference)
import jax
import jax.numpy as jnp
from jax import lax
from jax.experimental import pallas as pl
from jax.experimental.pallas import tpu as pltpu
from jax.experimental.pallas import tpu_sc as plsc

VOCAB = 1000000
MAXLEN = 200
EMBED = 32
BATCH = 4096

NC = 2
NS = 16
NW = NC * NS
LT = MAXLEN // 8
BT = BATCH // 128
ET = EMBED // 8
NQ = 8
LSPAN = MAXLEN // 4
LTSPAN = LSPAN // 8 + 2


def _body(x_hbm, tok_hbm, pos_hbm, out_hbm,
          idx_v, gbuf0, gbuf1, obuf0, obuf1, pos_v,
          sg0, sg1, ss0, ss1, si):
    gbuf = (gbuf0, gbuf1)
    obuf = (obuf0, obuf1)
    sg = (sg0, sg1)
    ss = (ss0, ss1)
    w = lax.axis_index("s") * NC + lax.axis_index("c")
    q = w // 4
    l0 = (w % 4) * LSPAN
    lt0 = l0 // 8

    pltpu.sync_copy(pos_hbm, pos_v)
    for t in range(LTSPAN):
        for j in range(4):
            pltpu.async_copy(x_hbm.at[lt0 + t, 4 * q + j], idx_v.at[t, j], si)
    pltpu.make_async_copy(x_hbm.at[0, 0], idx_v.at[0, 0], si).wait()
    for _ in range(LTSPAN * 4 - 1):
        pltpu.make_async_copy(x_hbm.at[0, 0], idx_v.at[0, 0], si).wait()

    iota = lax.iota(jnp.int32, 16)
    pat = (
        (iota // 8) * 4096 + (iota % 8) * 128,
        ((iota + 16) // 8) * 4096 + ((iota + 16) % 8) * 128,
    )

    def fire_gathers(l, b):
        t = l // 8 - lt0
        ls = l % 8
        for j in range(4):
            pltpu.async_copy(tok_hbm.at[idx_v.at[t, j, ls]], gbuf[b].at[j], sg[b])

    fire_gathers(l0, 0)

    @pl.loop(0, LSPAN, step=2)
    def _seq(g):
        for b in range(2):
            l = l0 + g + b
            o = 1 - b

            @pl.when(g + b + 1 < LSPAN)
            def _():
                fire_gathers(l + 1, o)

            for j in range(4):
                pltpu.make_async_copy(
                    tok_hbm.at[pl.ds(0, 128)], gbuf[b].at[j], sg[b]).wait()

            @pl.when(g + b >= 2)
            def _():
                for _ in range(ET):
                    pltpu.make_async_copy(
                        obuf[b].at[pl.ds(0, 4096)], out_hbm.at[0, 0, 0], ss[b]).wait()

            ph = (pos_v.at[l, pl.ds(0, 16)][...], pos_v.at[l, pl.ds(16, 16)][...])

            for btj in range(4):
                base = btj * 1024

                @pl.loop(0, 128, unroll=4)
                def _row(bs):
                    for h in range(2):
                        v = gbuf[b].at[btj, bs, pl.ds(h * 16, 16)][...] + ph[h]
                        plsc.store_scatter(obuf[b], [pat[h] + (base + bs)], v)

            for et in range(ET):
                pltpu.async_copy(
                    obuf[b].at[pl.ds(et * 4096, 4096)], out_hbm.at[l, et, q], ss[b])

    for b in range(2):
        for _ in range(ET):
            pltpu.make_async_copy(
                obuf[b].at[pl.ds(0, 4096)], out_hbm.at[0, 0, 0], ss[b]).wait()


def kernel(x, token_table, pos_table):
    x4 = jnp.transpose(
        jnp.reshape(jnp.transpose(x), (LT, 8, BT, 128)), (0, 2, 1, 3))
    mesh = plsc.VectorSubcoreMesh(core_axis_name="c", subcore_axis_name="s")
    k = pl.kernel(
        _body,
        out_type=jax.ShapeDtypeStruct((MAXLEN, ET, NQ, 4096), jnp.float32),
        mesh=mesh,
        compiler_params=pltpu.CompilerParams(
            use_tc_tiling_on_sc=False, needs_layout_passes=False),
        scratch_types=[
            pltpu.VMEM((LTSPAN, 4, 8, 128), jnp.int32),
            pltpu.VMEM((4, 128, EMBED), jnp.float32),
            pltpu.VMEM((4, 128, EMBED), jnp.float32),
            pltpu.VMEM((ET * 4096,), jnp.float32),
            pltpu.VMEM((ET * 4096,), jnp.float32),
            pltpu.VMEM((MAXLEN, EMBED), jnp.float32),
            pltpu.SemaphoreType.DMA,
            pltpu.SemaphoreType.DMA,
            pltpu.SemaphoreType.DMA,
            pltpu.SemaphoreType.DMA,
            pltpu.SemaphoreType.DMA,
        ],
    )
    out5 = k(x4, token_table, pos_table)
    return jnp.reshape(
        jnp.transpose(
            jnp.reshape(out5, (MAXLEN, ET, BT, 8, 128)), (2, 4, 0, 1, 3)),
        (BATCH, MAXLEN, EMBED))


# speedup vs baseline: 1.3739x; 1.3739x over previous
import jax
import jax.numpy as jnp
from jax import lax
from jax.experimental import pallas as pl
from jax.experimental.pallas import tpu as pltpu
from jax.experimental.pallas import tpu_sc as plsc

VOCAB = 1000000
MAXLEN = 200
EMBED = 32
BATCH = 4096

NC = 2
NS = 16
NW = NC * NS
LT = MAXLEN // 8
BT = BATCH // 128
ET = EMBED // 8
NQ = 8
LSPAN = MAXLEN // 4
LTSPAN = LSPAN // 8 + 2


def _body(x_hbm, tok_hbm, pos_hbm, out_hbm,
          idx_v, gbuf0, gbuf1, obuf0, obuf1, pos_v,
          sg0, sg1, ss0, ss1, si):
    gbuf = (gbuf0, gbuf1)
    obuf = (obuf0, obuf1)
    sg = (sg0, sg1)
    ss = (ss0, ss1)
    w = lax.axis_index("s") * NC + lax.axis_index("c")
    q = w // 4
    l0 = (w % 4) * LSPAN
    lt0 = l0 // 8

    pltpu.sync_copy(pos_hbm, pos_v)
    for t in range(LTSPAN):
        for j in range(4):
            pltpu.async_copy(x_hbm.at[lt0 + t, 4 * q + j], idx_v.at[t, j], si)
    pltpu.make_async_copy(x_hbm.at[0, 0], idx_v.at[0, 0], si).wait()
    for _ in range(LTSPAN * 4 - 1):
        pltpu.make_async_copy(x_hbm.at[0, 0], idx_v.at[0, 0], si).wait()

    iota = lax.iota(jnp.int32, 16)
    pat = (iota, iota + 16)
    zero16 = iota * 0

    def fire_gathers(l, b):
        t = l // 8 - lt0
        ls = l % 8
        for j in range(4):
            pltpu.async_copy(tok_hbm.at[idx_v.at[t, j, ls]], gbuf[b].at[j], sg[b])

    fire_gathers(l0, 0)

    @pl.loop(0, LSPAN, step=2)
    def _seq(g):
        for b in range(2):
            l = l0 + g + b
            o = 1 - b

            @pl.when(g + b + 1 < LSPAN)
            def _():
                fire_gathers(l + 1, o)

            for j in range(4):
                pltpu.make_async_copy(
                    tok_hbm.at[pl.ds(0, 128)], gbuf[b].at[j], sg[b]).wait()

            @pl.when(g + b >= 2)
            def _():
                for _ in range(ET * 4):
                    pltpu.make_async_copy(
                        obuf[b].at[pl.ds(0, 8), pl.ds(0, 128)],
                        out_hbm.at[0, 0, 0, 0], ss[b]).wait()

            ph = (pos_v.at[l, pl.ds(0, 16)][...], pos_v.at[l, pl.ds(16, 16)][...])

            for btj in range(4):
                base = btj * 128

                @pl.loop(0, 128, unroll=4)
                def _row(bs):
                    for h in range(2):
                        v = gbuf[b].at[btj, bs, pl.ds(h * 16, 16)][...] + ph[h]
                        plsc.store_scatter(
                            obuf[b], [pat[h], zero16 + (base + bs)], v)

            for et in range(ET):
                for btj in range(4):
                    pltpu.async_copy(
                        obuf[b].at[pl.ds(et * 8, 8), pl.ds(btj * 128, 128)],
                        out_hbm.at[l, et, q, btj], ss[b])

    for b in range(2):
        for _ in range(ET * 4):
            pltpu.make_async_copy(
                obuf[b].at[pl.ds(0, 8), pl.ds(0, 128)],
                out_hbm.at[0, 0, 0, 0], ss[b]).wait()


def kernel(x, token_table, pos_table):
    x4 = jnp.transpose(
        jnp.reshape(jnp.transpose(x), (LT, 8, BT, 128)), (0, 2, 1, 3))
    mesh = plsc.VectorSubcoreMesh(core_axis_name="c", subcore_axis_name="s")
    k = pl.kernel(
        _body,
        out_type=jax.ShapeDtypeStruct((MAXLEN, ET, NQ, 4, 8, 128), jnp.float32),
        mesh=mesh,
        compiler_params=pltpu.CompilerParams(
            use_tc_tiling_on_sc=False, needs_layout_passes=False),
        scratch_types=[
            pltpu.VMEM((LTSPAN, 4, 8, 128), jnp.int32),
            pltpu.VMEM((4, 128, EMBED), jnp.float32),
            pltpu.VMEM((4, 128, EMBED), jnp.float32),
            pltpu.VMEM((32, 513), jnp.float32),
            pltpu.VMEM((32, 513), jnp.float32),
            pltpu.VMEM((MAXLEN, EMBED), jnp.float32),
            pltpu.SemaphoreType.DMA,
            pltpu.SemaphoreType.DMA,
            pltpu.SemaphoreType.DMA,
            pltpu.SemaphoreType.DMA,
            pltpu.SemaphoreType.DMA,
        ],
    )
    out5 = k(x4, token_table, pos_table)
    return jnp.reshape(
        jnp.transpose(
            jnp.reshape(out5, (MAXLEN, ET, BT, 8, 128)), (2, 4, 0, 1, 3)),
        (BATCH, MAXLEN, EMBED))


# speedup vs baseline: 1.3753x; 1.0010x over previous
import jax
import jax.numpy as jnp
from jax import lax
from jax.experimental import pallas as pl
from jax.experimental.pallas import tpu as pltpu
from jax.experimental.pallas import tpu_sc as plsc

VOCAB = 1000000
MAXLEN = 200
EMBED = 32
BATCH = 4096

NC = 2
NS = 16
NW = NC * NS
LT = MAXLEN // 8
BT = BATCH // 128
ET = EMBED // 8
NQ = 8
LSPAN = MAXLEN // 4
LTSPAN = LSPAN // 8 + 2


def _body(x_hbm, tok_hbm, pos_hbm, out_hbm,
          idx_v, gbuf0, gbuf1, obuf0, obuf1, pos_v,
          sg0, sg1, ss0, ss1, si):
    gbuf = (gbuf0, gbuf1)
    obuf = (obuf0, obuf1)
    sg = (sg0, sg1)
    ss = (ss0, ss1)
    w = lax.axis_index("s") * NC + lax.axis_index("c")
    q = w // 4
    l0 = (w % 4) * LSPAN
    lt0 = l0 // 8

    pltpu.sync_copy(pos_hbm, pos_v)
    for t in range(LTSPAN):
        for j in range(4):
            pltpu.async_copy(x_hbm.at[lt0 + t, 4 * q + j], idx_v.at[t, j], si)
    for _ in range(LTSPAN * 4):
        pltpu.make_async_copy(x_hbm.at[0, 0], idx_v.at[0, 0], si).wait()

    iota = lax.iota(jnp.int32, 16)
    rows = (iota, iota + 16)
    zero16 = iota * 0

    def fire_gathers(l, b):
        t = l // 8 - lt0
        ls = l % 8
        for j in range(4):
            pltpu.async_copy(tok_hbm.at[idx_v.at[t, j, ls]], gbuf[b].at[j], sg[b])

    fire_gathers(l0, 0)

    @pl.loop(0, LSPAN, step=2)
    def _seq(g):
        for b in range(2):
            l = l0 + g + b
            o = 1 - b

            @pl.when(g + b + 1 < LSPAN)
            def _():
                fire_gathers(l + 1, o)

            for j in range(4):
                pltpu.make_async_copy(
                    tok_hbm.at[pl.ds(0, 128)], gbuf[b].at[j], sg[b]).wait()

            @pl.when(g + b >= 2)
            def _():
                for _ in range(ET):
                    pltpu.make_async_copy(
                        obuf[b].at[:, pl.ds(0, 8), pl.ds(0, 128)],
                        out_hbm.at[0, 0, 0], ss[b]).wait()

            ph = (pos_v.at[l, pl.ds(0, 16)][...], pos_v.at[l, pl.ds(16, 16)][...])

            for btj in range(4):
                @pl.loop(0, 128, unroll=8)
                def _row(bs):
                    cols = zero16 + bs
                    for h in range(2):
                        v = gbuf[b].at[btj, bs, pl.ds(h * 16, 16)][...] + ph[h]
                        plsc.store_scatter(
                            obuf[b], [zero16 + btj, rows[h], cols], v)

            for et in range(ET):
                pltpu.async_copy(
                    obuf[b].at[:, pl.ds(et * 8, 8), pl.ds(0, 128)],
                    out_hbm.at[l, et, q], ss[b])

    for b in range(2):
        for _ in range(ET):
            pltpu.make_async_copy(
                obuf[b].at[:, pl.ds(0, 8), pl.ds(0, 128)],
                out_hbm.at[0, 0, 0], ss[b]).wait()


def kernel(x, token_table, pos_table):
    x4 = jnp.transpose(
        jnp.reshape(jnp.transpose(x), (LT, 8, BT, 128)), (0, 2, 1, 3))
    mesh = plsc.VectorSubcoreMesh(core_axis_name="c", subcore_axis_name="s")
    k = pl.kernel(
        _body,
        out_type=jax.ShapeDtypeStruct((MAXLEN, ET, NQ, 4, 8, 128), jnp.float32),
        mesh=mesh,
        compiler_params=pltpu.CompilerParams(
            use_tc_tiling_on_sc=False, needs_layout_passes=False),
        scratch_types=[
            pltpu.VMEM((LTSPAN, 4, 8, 128), jnp.int32),
            pltpu.VMEM((4, 128, EMBED), jnp.float32),
            pltpu.VMEM((4, 128, EMBED), jnp.float32),
            pltpu.VMEM((4, 32, 129), jnp.float32),
            pltpu.VMEM((4, 32, 129), jnp.float32),
            pltpu.VMEM((MAXLEN, EMBED), jnp.float32),
            pltpu.SemaphoreType.DMA,
            pltpu.SemaphoreType.DMA,
            pltpu.SemaphoreType.DMA,
            pltpu.SemaphoreType.DMA,
            pltpu.SemaphoreType.DMA,
        ],
    )
    out6 = k(x4, token_table, pos_table)
    return jnp.reshape(
        jnp.transpose(
            jnp.reshape(out6, (MAXLEN, ET, BT, 8, 128)), (2, 4, 0, 1, 3)),
        (BATCH, MAXLEN, EMBED))


# speedup vs baseline: 1.8157x; 1.3202x over previous
import jax
import jax.numpy as jnp
from jax import lax
from jax.experimental import pallas as pl
from jax.experimental.pallas import tpu as pltpu
from jax.experimental.pallas import tpu_sc as plsc

VOCAB = 1000000
MAXLEN = 200
EMBED = 32
BATCH = 4096

NC = 2
NS = 16
NW = NC * NS
LT = MAXLEN // 8
BT = BATCH // 128
ET = EMBED // 8
NQ = 8
LSPAN = MAXLEN // 4
LTSPAN = LSPAN // 8 + 2


def _body(x_hbm, tok_hbm, pos_hbm, out_hbm,
          idx_v, gbuf0, gbuf1, obuf0, obuf1, pos_v,
          sg0, sg1, ss0, ss1, si):
    gbuf = (gbuf0, gbuf1)
    obuf = (obuf0, obuf1)
    sg = (sg0, sg1)
    ss = (ss0, ss1)
    w = lax.axis_index("s") * NC + lax.axis_index("c")
    q = w // 4
    l0 = (w % 4) * LSPAN
    lt0 = l0 // 8

    pltpu.sync_copy(pos_hbm, pos_v)
    for t in range(LTSPAN):
        for j in range(4):
            pltpu.async_copy(x_hbm.at[lt0 + t, 4 * q + j], idx_v.at[t, j], si)
    for _ in range(LTSPAN * 4):
        pltpu.make_async_copy(x_hbm.at[0, 0], idx_v.at[0, 0], si).wait()

    iota = lax.iota(jnp.int32, 16)
    rows = (iota, iota + 16)
    zero16 = iota * 0

    def fire_gathers(l, b):
        t = l // 8 - lt0
        ls = l % 8
        for j in range(4):
            pltpu.async_copy(tok_hbm.at[idx_v.at[t, j, ls]], gbuf[b].at[j], sg[b])

    fire_gathers(l0, 0)

    @pl.loop(0, LSPAN, step=2)
    def _seq(g):
        for b in range(2):
            l = l0 + g + b
            o = 1 - b

            @pl.when(g + b + 1 < LSPAN)
            def _():
                fire_gathers(l + 1, o)

            for j in range(4):
                pltpu.make_async_copy(
                    tok_hbm.at[pl.ds(0, 128)], gbuf[b].at[j], sg[b]).wait()

            @pl.when(g + b >= 2)
            def _():
                for _ in range(ET):
                    pltpu.make_async_copy(
                        obuf[b].at[:, pl.ds(0, 8), pl.ds(0, 128)],
                        out_hbm.at[0, 0, 0], ss[b]).wait()

            ph = (pos_v.at[l, pl.ds(0, 16)][...], pos_v.at[l, pl.ds(16, 16)][...])

            for btj in range(4):
                @plsc.parallel_loop(0, 128, step=1, unroll=8)
                def _row(bs):
                    cols = zero16 + bs
                    for h in range(2):
                        v = gbuf[b].at[btj, bs, pl.ds(h * 16, 16)][...] + ph[h]
                        plsc.store_scatter(
                            obuf[b], [zero16 + btj, rows[h], cols], v)

            for et in range(ET):
                pltpu.async_copy(
                    obuf[b].at[:, pl.ds(et * 8, 8), pl.ds(0, 128)],
                    out_hbm.at[l, et, q], ss[b])

    for b in range(2):
        for _ in range(ET):
            pltpu.make_async_copy(
                obuf[b].at[:, pl.ds(0, 8), pl.ds(0, 128)],
                out_hbm.at[0, 0, 0], ss[b]).wait()


def kernel(x, token_table, pos_table):
    x4 = jnp.transpose(
        jnp.reshape(jnp.transpose(x), (LT, 8, BT, 128)), (0, 2, 1, 3))
    mesh = plsc.VectorSubcoreMesh(core_axis_name="c", subcore_axis_name="s")
    k = pl.kernel(
        _body,
        out_type=jax.ShapeDtypeStruct((MAXLEN, ET, NQ, 4, 8, 128), jnp.float32),
        mesh=mesh,
        compiler_params=pltpu.CompilerParams(
            use_tc_tiling_on_sc=False, needs_layout_passes=False),
        scratch_types=[
            pltpu.VMEM((LTSPAN, 4, 8, 128), jnp.int32),
            pltpu.VMEM((4, 128, EMBED), jnp.float32),
            pltpu.VMEM((4, 128, EMBED), jnp.float32),
            pltpu.VMEM((4, 32, 129), jnp.float32),
            pltpu.VMEM((4, 32, 129), jnp.float32),
            pltpu.VMEM((MAXLEN, EMBED), jnp.float32),
            pltpu.SemaphoreType.DMA,
            pltpu.SemaphoreType.DMA,
            pltpu.SemaphoreType.DMA,
            pltpu.SemaphoreType.DMA,
            pltpu.SemaphoreType.DMA,
        ],
    )
    out6 = k(x4, token_table, pos_table)
    return jnp.reshape(
        jnp.transpose(
            jnp.reshape(out6, (MAXLEN, ET, BT, 8, 128)), (2, 4, 0, 1, 3)),
        (BATCH, MAXLEN, EMBED))
